# baseline (speedup 1.0000x reference)
import jax
import jax.numpy as jnp
from jax import lax
from jax.experimental import pallas as pl
from jax.experimental.pallas import tpu as pltpu

_BR = 80
_NB = 10


def _gcn_body(x_ref, adj_ref, w1_ref, w2_ref, wout_ref,
              x3_ref, y_ref, abuf, xbuf, s1_ref, s2_ref, sem, xsem):
    n = xbuf.shape[0]
    r = n // _BR
    total = 2 * r

    def block_copy(k, b):
        blk = lax.rem(k, r)
        return pltpu.make_async_copy(
            adj_ref.at[pl.ds(blk * _BR, _BR), :], abuf.at[b], sem.at[b])

    block_copy(0, 0).start()
    block_copy(1, 1).start()
    x_copy = pltpu.make_async_copy(x_ref, xbuf, xsem)
    x_copy.start()
    for b in range(2, _NB):
        block_copy(b, b).start()

    x_copy.wait()
    s1 = jnp.dot(xbuf[...], w1_ref[...], preferred_element_type=jnp.float32)
    s1_ref[...] = s1.astype(jnp.bfloat16)
    w2 = w2_ref[...].astype(jnp.bfloat16)
    wout = wout_ref[...].astype(jnp.bfloat16)

    def super_step(s, carry):
        for b in range(_NB):
            k = s * _NB + b
            row = lax.rem(k, r) * _BR
            block_copy(k, b).wait()
            a = abuf[b].astype(jnp.bfloat16)

            @pl.when(k + _NB < total)
            def _():
                block_copy(k + _NB, b).start()

            @pl.when(k < r)
            def _():
                h = jnp.dot(a, s1_ref[...], preferred_element_type=jnp.float32)
                x1 = jnp.where(h >= 0, h, 0.01 * h)
                s2 = jnp.dot(x1.astype(jnp.bfloat16), w2,
                             preferred_element_type=jnp.float32)
                s2_ref[pl.ds(row, _BR), :] = s2.astype(jnp.bfloat16)

            @pl.when(k >= r)
            def _():
                x3 = jnp.dot(a, s2_ref[...], preferred_element_type=jnp.float32)
                x3_ref[pl.ds(row, _BR), :] = x3
                logits = jnp.dot(x3.astype(jnp.bfloat16), wout,
                                 preferred_element_type=jnp.float32)
                y_ref[pl.ds(row, _BR), :] = jax.nn.sigmoid(logits)
        return carry

    lax.fori_loop(0, total // _NB, super_step, 0)


def kernel(x, adj, W1, W2, W_out):
    n, nfeat = x.shape
    nhid = W1.shape[1]
    nclass = W_out.shape[1]

    x3, y = pl.pallas_call(
        _gcn_body,
        in_specs=[
            pl.BlockSpec(memory_space=pl.ANY),
            pl.BlockSpec(memory_space=pl.ANY),
            pl.BlockSpec(memory_space=pltpu.MemorySpace.VMEM),
            pl.BlockSpec(memory_space=pltpu.MemorySpace.VMEM),
            pl.BlockSpec(memory_space=pltpu.MemorySpace.VMEM),
        ],
        out_specs=[
            pl.BlockSpec(memory_space=pltpu.MemorySpace.VMEM),
            pl.BlockSpec(memory_space=pltpu.MemorySpace.VMEM),
        ],
        out_shape=[
            jax.ShapeDtypeStruct((n, nhid), jnp.float32),
            jax.ShapeDtypeStruct((n, nclass), jnp.float32),
        ],
        scratch_shapes=[
            pltpu.VMEM((_NB, _BR, n), jnp.float32),
            pltpu.VMEM((n, nfeat), jnp.float32),
            pltpu.VMEM((n, nhid), jnp.bfloat16),
            pltpu.VMEM((n, nhid), jnp.bfloat16),
            pltpu.SemaphoreType.DMA((_NB,)),
            pltpu.SemaphoreType.DMA,
        ],
    )(x, adj, W1, W2, W_out)

    return (y, x3)


# speedup vs baseline: 1.0195x; 1.0195x over previous
import jax
import jax.numpy as jnp
from jax import lax
from jax.experimental import pallas as pl
from jax.experimental.pallas import tpu as pltpu

_BR = 80
_NB = 5


def _gcn_body(x_ref, adj_ref, w1_ref, w2_ref, wout_ref,
              x3_ref, y_ref, abuf, xbuf, s1_ref, s2_ref, sem, xsem):
    n = xbuf.shape[0]
    r = n // _BR
    total = 2 * r

    def block_copy(k, b):
        blk = lax.rem(k, r)
        return pltpu.make_async_copy(
            adj_ref.at[pl.ds(blk * _BR, _BR), :], abuf.at[b], sem.at[b])

    block_copy(0, 0).start()
    block_copy(1, 1).start()
    x_copy = pltpu.make_async_copy(x_ref, xbuf, xsem)
    x_copy.start()
    for b in range(2, _NB):
        block_copy(b, b).start()

    x_copy.wait()
    s1 = jnp.dot(xbuf[...], w1_ref[...], preferred_element_type=jnp.float32)
    s1_ref[...] = s1.astype(jnp.bfloat16)
    w2 = w2_ref[...].astype(jnp.bfloat16)
    wout = wout_ref[...].astype(jnp.bfloat16)

    def super_step(s, carry):
        for b in range(_NB):
            k = s * _NB + b
            row = lax.rem(k, r) * _BR
            block_copy(k, b).wait()
            a = abuf[b].astype(jnp.bfloat16)

            @pl.when(k + _NB < total)
            def _():
                block_copy(k + _NB, b).start()

            @pl.when(k < r)
            def _():
                h = jnp.dot(a, s1_ref[...], preferred_element_type=jnp.float32)
                x1 = jnp.where(h >= 0, h, 0.01 * h)
                s2 = jnp.dot(x1.astype(jnp.bfloat16), w2,
                             preferred_element_type=jnp.float32)
                s2_ref[pl.ds(row, _BR), :] = s2.astype(jnp.bfloat16)

            @pl.when(k >= r)
            def _():
                x3 = jnp.dot(a, s2_ref[...], preferred_element_type=jnp.float32)
                x3_ref[pl.ds(row, _BR), :] = x3
                logits = jnp.dot(x3.astype(jnp.bfloat16), wout,
                                 preferred_element_type=jnp.float32)
                y_ref[pl.ds(row, _BR), :] = jax.nn.sigmoid(logits)
        return carry

    lax.fori_loop(0, total // _NB, super_step, 0)


def kernel(x, adj, W1, W2, W_out):
    n, nfeat = x.shape
    nhid = W1.shape[1]
    nclass = W_out.shape[1]

    x3, y = pl.pallas_call(
        _gcn_body,
        in_specs=[
            pl.BlockSpec(memory_space=pl.ANY),
            pl.BlockSpec(memory_space=pl.ANY),
            pl.BlockSpec(memory_space=pltpu.MemorySpace.VMEM),
            pl.BlockSpec(memory_space=pltpu.MemorySpace.VMEM),
            pl.BlockSpec(memory_space=pltpu.MemorySpace.VMEM),
        ],
        out_specs=[
            pl.BlockSpec(memory_space=pltpu.MemorySpace.VMEM),
            pl.BlockSpec(memory_space=pltpu.MemorySpace.VMEM),
        ],
        out_shape=[
            jax.ShapeDtypeStruct((n, nhid), jnp.float32),
            jax.ShapeDtypeStruct((n, nclass), jnp.float32),
        ],
        scratch_shapes=[
            pltpu.VMEM((_NB, _BR, n), jnp.float32),
            pltpu.VMEM((n, nfeat), jnp.float32),
            pltpu.VMEM((n, nhid), jnp.bfloat16),
            pltpu.VMEM((n, nhid), jnp.bfloat16),
            pltpu.SemaphoreType.DMA((_NB,)),
            pltpu.SemaphoreType.DMA,
        ],
    )(x, adj, W1, W2, W_out)

    return (y, x3)
